# Initial kernel scaffold; baseline (speedup 1.0000x reference)
import jax
import jax.numpy as jnp
from jax.experimental import pallas as pl


def kernel(x, edge_index, W1, b1, W2, b2):
    raise NotImplementedError("write your pallas kernel here")


# speedup vs baseline: 4.3924x; 4.3924x over previous
import functools

import jax
import jax.numpy as jnp
from jax import lax
from jax.experimental import pallas as pl
from jax.experimental.pallas import tpu as pltpu
from jax.experimental.pallas import tpu_sc as plsc

N_NODES = 10000
HIDDEN = 128
N_EDGES = 320000
DROPOUT_RATE = 0.25

NC = 2
NS = 16
NW = NC * NS
CHUNK = 128
EPW_CHUNKS = -(-N_EDGES // (NW * CHUNK))
EPW = EPW_CHUNKS * CHUNK
E_PAD = NW * EPW
ACC_ROWS = 10240
ZROWS = 64
ROWS_PER_SUB = ACC_ROWS // NS


def _sc_scatter(x, src, dst):
    mesh = plsc.VectorSubcoreMesh(core_axis_name="c", subcore_axis_name="s")

    @functools.partial(
        pl.kernel,
        mesh=mesh,
        out_type=jax.ShapeDtypeStruct((NC, ACC_ROWS, HIDDEN), jnp.float32),
        scratch_types=[
            pltpu.VMEM((CHUNK,), jnp.int32),
            pltpu.VMEM((CHUNK,), jnp.int32),
            pltpu.VMEM((CHUNK, HIDDEN), jnp.float32),
            pltpu.VMEM((ZROWS, HIDDEN), jnp.float32),
            pltpu.VMEM_SHARED((ACC_ROWS, HIDDEN), jnp.float32),
            pltpu.SemaphoreType.DMA,
        ],
    )
    def k(x_hbm, src_hbm, dst_hbm, out_hbm, src_v, dst_v, rows_v, z_v, acc_sh, sem):
        c = lax.axis_index("c")
        s = lax.axis_index("s")
        wid = c * NS + s

        def zrow(i, carry):
            for j in range(HIDDEN // 16):
                z_v[i, pl.ds(j * 16, 16)] = jnp.zeros((16,), jnp.float32)
            return carry

        lax.fori_loop(0, ZROWS, zrow, 0)

        def zacc(i, carry):
            pltpu.sync_copy(z_v, acc_sh.at[pl.ds(s * ROWS_PER_SUB + i * ZROWS, ZROWS)])
            return carry

        lax.fori_loop(0, ROWS_PER_SUB // ZROWS, zacc, 0)
        plsc.subcore_barrier()

        base = wid * EPW

        def body(j, carry):
            eb = pl.multiple_of(base + j * CHUNK, CHUNK)
            pltpu.sync_copy(src_hbm.at[pl.ds(eb, CHUNK)], src_v)
            pltpu.sync_copy(dst_hbm.at[pl.ds(eb, CHUNK)], dst_v)
            pltpu.async_copy(x_hbm.at[src_v], rows_v, sem).wait()
            pltpu.sync_copy(rows_v, acc_sh.at[dst_v], add=True)
            return carry

        lax.fori_loop(0, EPW_CHUNKS, body, 0)
        plsc.subcore_barrier()

        pltpu.sync_copy(
            acc_sh.at[pl.ds(s * ROWS_PER_SUB, ROWS_PER_SUB)],
            out_hbm.at[c, pl.ds(s * ROWS_PER_SUB, ROWS_PER_SUB)],
        )

    return k(x, src, dst)


def _tc_mlp(x, p0, p1, mask, W1, b1, W2, b2):
    BLK = 1000

    def body(x_ref, p0_ref, p1_ref, m_ref, w1_ref, b1_ref, w2_ref, b2_ref, o_ref):
        h = (x_ref[...] + p0_ref[...] + p1_ref[...]) * m_ref[...]
        h = jnp.dot(h, w1_ref[...], preferred_element_type=jnp.float32) + b1_ref[...]
        h = jnp.maximum(h, 0.0)
        o_ref[...] = jnp.dot(h, w2_ref[...], preferred_element_type=jnp.float32) + b2_ref[...]

    row_spec = pl.BlockSpec((BLK, HIDDEN), lambda i: (i, 0))
    full_spec = pl.BlockSpec((HIDDEN, HIDDEN), lambda i: (0, 0))
    vec_spec = pl.BlockSpec((1, HIDDEN), lambda i: (0, 0))
    return pl.pallas_call(
        body,
        grid=(N_NODES // BLK,),
        in_specs=[row_spec, row_spec, row_spec, vec_spec, full_spec, vec_spec,
                  full_spec, vec_spec],
        out_specs=row_spec,
        out_shape=jax.ShapeDtypeStruct((N_NODES, HIDDEN), jnp.float32),
    )(x, p0, p1, mask, W1, b1, W2, b2)


def kernel(x, edge_index, W1, b1, W2, b2):
    mask = jax.random.bernoulli(
        jax.random.key(42), p=1.0 - DROPOUT_RATE, shape=(HIDDEN,)
    ).astype(x.dtype)
    src = edge_index[0].astype(jnp.int32)
    dst = edge_index[1].astype(jnp.int32)
    pad = E_PAD - N_EDGES
    src = jnp.concatenate([src, jnp.zeros((pad,), jnp.int32)])
    dst = jnp.concatenate([dst, jnp.full((pad,), N_NODES, jnp.int32)])
    partials = _sc_scatter(x, src, dst)
    return _tc_mlp(
        x, partials[0, :N_NODES], partials[1, :N_NODES], mask.reshape(1, HIDDEN),
        W1, b1.reshape(1, HIDDEN), W2, b2.reshape(1, HIDDEN),
    )
